# baseline (speedup 1.0000x reference)
import jax
import jax.numpy as jnp
from jax import lax
from jax.experimental import pallas as pl
from jax.experimental.pallas import tpu as pltpu

_N_DIMS = 15
_C = 2 ** _N_DIMS
_SUB = _C // 128


def _threefry2x32_i32(x0, x1, k1, k2):
    rot_a = (13, 15, 26, 6)
    rot_b = (17, 29, 16, 24)
    ks = (jnp.int32(k1), jnp.int32(k2),
          jnp.int32((k1 ^ k2 ^ 0x1BD11BDA) - (1 << 32)
                    if (k1 ^ k2 ^ 0x1BD11BDA) >= (1 << 31)
                    else (k1 ^ k2 ^ 0x1BD11BDA)))
    x = [x0 + ks[0], x1 + ks[1]]

    def rnd(v, r):
        v0 = v[0] + v[1]
        v1 = lax.shift_left(v[1], jnp.int32(r)) | lax.shift_right_logical(
            v[1], jnp.int32(32 - r))
        return [v0, v0 ^ v1]

    for i in range(5):
        for r in (rot_a if i % 2 == 0 else rot_b):
            x = rnd(x, r)
        x = [x[0] + ks[(i + 1) % 3],
             x[1] + ks[(i + 2) % 3] + jnp.int32(i + 1)]
    return x


def _fused_kernel(lp_ref, x_ref, pow2_ref, y_ref, nlp_ref):
    r = pl.program_id(0)

    sub = lax.broadcasted_iota(jnp.int32, (_SUB, 128), 0)
    lane = lax.broadcasted_iota(jnp.int32, (_SUB, 128), 1)
    c = sub * 128 + lane
    lo = r * _C + c
    o = _threefry2x32_i32(jnp.zeros_like(lo), lo, 0, 42)
    v = lax.shift_right_logical(o[0] ^ o[1], jnp.int32(9))
    m = jnp.max(v)
    idx = jnp.min(jnp.where(v == m, c, jnp.int32(_C)))

    lp = lp_ref[...]
    mlp = jnp.max(lp)
    logz = mlp + jnp.log(jnp.sum(jnp.exp(lp - mlp)))
    lp_idx = jnp.sum(jnp.where(c == idx, lp, 0.0))
    nlp_ref[...] = jnp.broadcast_to(logz - lp_idx, (1, 1, 1))

    sign = jnp.where((pow2_ref[...] & idx) != 0, -1.0, 1.0)
    y_ref[...] = x_ref[...] * sign


def kernel(x, flip_log_prob, flip_dirs):
    del flip_dirs
    b = x.shape[0]
    xr = x.reshape(b, 1024, 128)
    lp = flip_log_prob.reshape(_SUB, 128)
    pow2 = jnp.asarray([1 << (l % 16) for l in range(128)],
                       dtype=jnp.int32).reshape(1, 1, 128)

    y, nlp = pl.pallas_call(
        _fused_kernel,
        grid=(b,),
        in_specs=[
            pl.BlockSpec((_SUB, 128), lambda r: (0, 0)),
            pl.BlockSpec((1, 1024, 128), lambda r: (r, 0, 0)),
            pl.BlockSpec((1, 1, 128), lambda r: (0, 0, 0)),
        ],
        out_specs=[
            pl.BlockSpec((1, 1024, 128), lambda r: (r, 0, 0)),
            pl.BlockSpec((1, 1, 1), lambda r: (r, 0, 0)),
        ],
        out_shape=[
            jax.ShapeDtypeStruct((b, 1024, 128), x.dtype),
            jax.ShapeDtypeStruct((b, 1, 1), jnp.float32),
        ],
        compiler_params=pltpu.CompilerParams(
            dimension_semantics=("parallel",)),
    )(lp, xr, pow2)

    return (y.reshape(x.shape), nlp.reshape(b))


# speedup vs baseline: 1.0251x; 1.0251x over previous
import jax
import jax.numpy as jnp
from jax import lax
from jax.experimental import pallas as pl
from jax.experimental.pallas import tpu as pltpu

_N_DIMS = 15
_C = 2 ** _N_DIMS
_SUB = _C // 128
_CHUNK = 64
_K2 = 42
_KS2 = 0x1BD11BDA ^ _K2


def _threefry_chunk(lo):
    rot_a = (13, 15, 26, 6)
    rot_b = (17, 29, 16, 24)
    ks = (jnp.int32(0), jnp.int32(_K2), jnp.int32(_KS2))
    x = [jnp.zeros_like(lo), lo + ks[1]]

    def rnd(v, r):
        v0 = v[0] + v[1]
        v1 = lax.shift_left(v[1], jnp.int32(r)) | lax.shift_right_logical(
            v[1], jnp.int32(32 - r))
        return [v0, v0 ^ v1]

    for i in range(5):
        for r in (rot_a if i % 2 == 0 else rot_b):
            x = rnd(x, r)
        x = [x[0] + ks[(i + 1) % 3],
             x[1] + ks[(i + 2) % 3] + jnp.int32(i + 1)]
    return lax.shift_right_logical(x[0] ^ x[1], jnp.int32(9))


def _fused_kernel(lp_ref, x_ref, pow2_ref, y_ref, nlp_ref, v_scr):
    r = pl.program_id(0)

    base = r * _C
    for k in range(_SUB // _CHUNK):
        sub = lax.broadcasted_iota(jnp.int32, (_CHUNK, 128), 0)
        lane = lax.broadcasted_iota(jnp.int32, (_CHUNK, 128), 1)
        lo = base + (k * _CHUNK + sub) * 128 + lane
        v_scr[k * _CHUNK:(k + 1) * _CHUNK, :] = _threefry_chunk(lo)

    v = v_scr[...]
    c = (lax.broadcasted_iota(jnp.int32, (_SUB, 128), 0) * 128
         + lax.broadcasted_iota(jnp.int32, (_SUB, 128), 1))
    m = jnp.max(v, axis=(0, 1), keepdims=True)
    idxv = jnp.min(jnp.where(v == m, c, jnp.int32(_C)),
                   axis=(0, 1), keepdims=True)

    lp = lp_ref[...]
    mlp = jnp.max(lp, axis=(0, 1), keepdims=True)
    logz = mlp + jnp.log(jnp.sum(jnp.exp(lp - mlp), axis=(0, 1),
                                 keepdims=True))
    lp_idx = jnp.sum(jnp.where(c == idxv, lp, 0.0), axis=(0, 1),
                     keepdims=True)
    nlp_ref[...] = (logz - lp_idx).reshape(1, 1, 1)

    sign = jnp.where((pow2_ref[...] & idxv.reshape(1, 1, 1)) != 0,
                     -1.0, 1.0)
    y_ref[...] = x_ref[...] * sign


def kernel(x, flip_log_prob, flip_dirs):
    del flip_dirs
    b = x.shape[0]
    xr = x.reshape(b, 1024, 128)
    lp = flip_log_prob.reshape(_SUB, 128)
    pow2 = jnp.asarray([1 << (l % 16) for l in range(128)],
                       dtype=jnp.int32).reshape(1, 1, 128)

    y, nlp = pl.pallas_call(
        _fused_kernel,
        grid=(b,),
        in_specs=[
            pl.BlockSpec((_SUB, 128), lambda r: (0, 0)),
            pl.BlockSpec((1, 1024, 128), lambda r: (r, 0, 0)),
            pl.BlockSpec((1, 1, 128), lambda r: (0, 0, 0)),
        ],
        out_specs=[
            pl.BlockSpec((1, 1024, 128), lambda r: (r, 0, 0)),
            pl.BlockSpec((1, 1, 1), lambda r: (r, 0, 0)),
        ],
        out_shape=[
            jax.ShapeDtypeStruct((b, 1024, 128), x.dtype),
            jax.ShapeDtypeStruct((b, 1, 1), jnp.float32),
        ],
        scratch_shapes=[pltpu.VMEM((_SUB, 128), jnp.int32)],
        compiler_params=pltpu.CompilerParams(
            dimension_semantics=("parallel",)),
    )(lp, xr, pow2)

    return (y.reshape(x.shape), nlp.reshape(b))


# speedup vs baseline: 1.1725x; 1.1439x over previous
import jax
import jax.numpy as jnp
from jax import lax
from jax.experimental import pallas as pl
from jax.experimental.pallas import tpu as pltpu

_N_DIMS = 15
_C = 2 ** _N_DIMS
_SUB = _C // 128
_CHUNK = 64
_ROWS = 8
_K2 = 42
_KS2 = 0x1BD11BDA ^ _K2


def _threefry_chunk(lo):
    rot_a = (13, 15, 26, 6)
    rot_b = (17, 29, 16, 24)
    ks = (jnp.int32(0), jnp.int32(_K2), jnp.int32(_KS2))
    x = [jnp.zeros_like(lo), lo + ks[1]]

    def rnd(v, r):
        v0 = v[0] + v[1]
        v1 = lax.shift_left(v[1], jnp.int32(r)) | lax.shift_right_logical(
            v[1], jnp.int32(32 - r))
        return [v0, v0 ^ v1]

    for i in range(5):
        for r in (rot_a if i % 2 == 0 else rot_b):
            x = rnd(x, r)
        x = [x[0] + ks[(i + 1) % 3],
             x[1] + ks[(i + 2) % 3] + jnp.int32(i + 1)]
    return lax.shift_right_logical(x[0] ^ x[1], jnp.int32(9))


def _fused_kernel(lp_ref, x_ref, pow2_ref, y_ref, nlp_ref, v_scr):
    g = pl.program_id(0)

    lp = lp_ref[...]
    c = (lax.broadcasted_iota(jnp.int32, (_SUB, 128), 0) * 128
         + lax.broadcasted_iota(jnp.int32, (_SUB, 128), 1))
    mlp = jnp.max(lp, axis=(0, 1), keepdims=True)
    logz = mlp + jnp.log(jnp.sum(jnp.exp(lp - mlp), axis=(0, 1),
                                 keepdims=True))

    nlp_acc = nlp_ref[...]
    out_lane = lax.broadcasted_iota(jnp.int32, (1, 128), 1)

    for i in range(_ROWS):
        base = (g * _ROWS + i) * _C
        for k in range(_SUB // _CHUNK):
            sub = lax.broadcasted_iota(jnp.int32, (_CHUNK, 128), 0)
            lane = lax.broadcasted_iota(jnp.int32, (_CHUNK, 128), 1)
            lo = base + (k * _CHUNK + sub) * 128 + lane
            v_scr[i, k * _CHUNK:(k + 1) * _CHUNK, :] = _threefry_chunk(lo)

        v = v_scr[i]
        m = jnp.max(v, axis=(0, 1), keepdims=True)
        idxv = jnp.min(jnp.where(v == m, c, jnp.int32(_C)),
                       axis=(0, 1), keepdims=True)

        lp_idx = jnp.sum(jnp.where(c == idxv, lp, 0.0), axis=(0, 1),
                         keepdims=True)
        nlp_acc = jnp.where(out_lane == g * _ROWS + i,
                            logz - lp_idx, nlp_acc)

        sign = jnp.where((pow2_ref[0] & idxv) != 0, -1.0, 1.0)
        y_ref[i] = x_ref[i] * sign

    nlp_ref[...] = nlp_acc


def kernel(x, flip_log_prob, flip_dirs):
    del flip_dirs
    b = x.shape[0]
    grid = b // _ROWS
    xr = x.reshape(b, 1024, 128)
    lp = flip_log_prob.reshape(_SUB, 128)
    pow2 = jnp.asarray([1 << (l % 16) for l in range(128)],
                       dtype=jnp.int32).reshape(1, 128)

    y, nlp = pl.pallas_call(
        _fused_kernel,
        grid=(grid,),
        in_specs=[
            pl.BlockSpec((_SUB, 128), lambda r: (0, 0)),
            pl.BlockSpec((_ROWS, 1024, 128), lambda r: (r, 0, 0)),
            pl.BlockSpec((1, 128), lambda r: (0, 0)),
        ],
        out_specs=[
            pl.BlockSpec((_ROWS, 1024, 128), lambda r: (r, 0, 0)),
            pl.BlockSpec((1, 128), lambda r: (0, 0)),
        ],
        out_shape=[
            jax.ShapeDtypeStruct((b, 1024, 128), x.dtype),
            jax.ShapeDtypeStruct((1, 128), jnp.float32),
        ],
        scratch_shapes=[pltpu.VMEM((_ROWS, _SUB, 128), jnp.int32)],
        compiler_params=pltpu.CompilerParams(
            dimension_semantics=("arbitrary",)),
    )(lp, xr, pow2)

    return (y.reshape(x.shape), nlp.reshape(b))
